# baseline (speedup 1.0000x reference)
import jax
import jax.numpy as jnp
from jax import lax
from jax.experimental import pallas as pl
from jax.experimental.pallas import tpu as pltpu
from jax.experimental.pallas import tpu_sc as plsc

B = 4096 * 200
D = 64
NC, NS = 2, 16
NW = NC * NS
BPW = B // NW
CHUNK = 1024
NIDX = 128
NGATH = CHUNK // NIDX
NCHUNKS = BPW // CHUNK


def _emb_body(x_hbm, table_hbm, out_hbm, idx_v, rows_v, sem):
    wid = lax.axis_index("s") * NC + lax.axis_index("c")
    wbase = wid * BPW

    def body(g, carry):
        base = wbase + g * CHUNK
        pltpu.sync_copy(x_hbm.at[pl.ds(base, CHUNK)], idx_v)
        descs = [
            pltpu.async_copy(
                table_hbm.at[idx_v.at[pl.ds(j * NIDX, NIDX)]],
                rows_v.at[pl.ds(j * NIDX, NIDX)],
                sem,
            )
            for j in range(NGATH)
        ]
        for d in descs:
            d.wait()
        pltpu.sync_copy(rows_v, out_hbm.at[pl.ds(base, CHUNK)])
        return carry

    lax.fori_loop(0, NCHUNKS, body, 0)


def kernel(x, table):
    xf = x.reshape(-1)
    mesh = plsc.VectorSubcoreMesh(core_axis_name="c", subcore_axis_name="s")
    out = pl.kernel(
        _emb_body,
        out_type=jax.ShapeDtypeStruct((B, D), jnp.float32),
        mesh=mesh,
        scratch_types=[
            pltpu.VMEM((CHUNK,), jnp.int32),
            pltpu.VMEM((CHUNK, D), jnp.float32),
            pltpu.SemaphoreType.DMA,
        ],
        compiler_params=pltpu.CompilerParams(use_tc_tiling_on_sc=False),
    )(xf, table)
    return out.reshape(x.shape + (D,))


# speedup vs baseline: 1.0139x; 1.0139x over previous
import jax
import jax.numpy as jnp
from jax import lax
from jax.experimental import pallas as pl
from jax.experimental.pallas import tpu as pltpu
from jax.experimental.pallas import tpu_sc as plsc

B = 4096 * 200
D = 64
NC, NS = 2, 16
NW = NC * NS
BPW = B // NW
CHUNK = 640
NIDX = 128
NGATH = CHUNK // NIDX
NCHUNKS = BPW // CHUNK


def _emb_body(x_hbm, table_hbm, out_hbm, idx_v, rows0, rows1, sem0, sem1):
    wid = lax.axis_index("s") * NC + lax.axis_index("c")
    wbase = wid * BPW
    pltpu.sync_copy(x_hbm.at[pl.ds(wbase, BPW)], idx_v)

    def fire(g, buf, sem):
        for j in range(NGATH):
            pltpu.async_copy(
                table_hbm.at[idx_v.at[pl.ds(g * CHUNK + j * NIDX, NIDX)]],
                buf.at[pl.ds(j * NIDX, NIDX)],
                sem,
            )

    def drain_write(g, buf, sem):
        pltpu.make_async_copy(table_hbm.at[pl.ds(0, CHUNK)], buf, sem).wait()
        pltpu.sync_copy(buf, out_hbm.at[pl.ds(wbase + g * CHUNK, CHUNK)])

    fire(0, rows0, sem0)

    def body(i2, carry):
        g0 = 2 * i2
        fire(g0 + 1, rows1, sem1)
        drain_write(g0, rows0, sem0)
        fire(g0 + 2, rows0, sem0)
        drain_write(g0 + 1, rows1, sem1)
        return carry

    lax.fori_loop(0, NCHUNKS // 2 - 1, body, 0)
    g0 = NCHUNKS - 2
    fire(g0 + 1, rows1, sem1)
    drain_write(g0, rows0, sem0)
    drain_write(g0 + 1, rows1, sem1)


def kernel(x, table):
    xf = x.reshape(-1)
    mesh = plsc.VectorSubcoreMesh(core_axis_name="c", subcore_axis_name="s")
    out = pl.kernel(
        _emb_body,
        out_type=jax.ShapeDtypeStruct((B, D), jnp.float32),
        mesh=mesh,
        scratch_types=[
            pltpu.VMEM((BPW,), jnp.int32),
            pltpu.VMEM((CHUNK, D), jnp.float32),
            pltpu.VMEM((CHUNK, D), jnp.float32),
            pltpu.SemaphoreType.DMA,
            pltpu.SemaphoreType.DMA,
        ],
        compiler_params=pltpu.CompilerParams(use_tc_tiling_on_sc=False),
    )(xf, table)
    return out.reshape(x.shape + (D,))


# speedup vs baseline: 1.2405x; 1.2235x over previous
import jax
import jax.numpy as jnp
from jax import lax
from jax.experimental import pallas as pl
from jax.experimental.pallas import tpu as pltpu
from jax.experimental.pallas import tpu_sc as plsc

B = 4096 * 200
D = 64
NC, NS = 2, 16
NW = NC * NS
BPW = B // NW
CHUNK = 256
NIDX = 128
NGATH = CHUNK // NIDX
NCHUNKS = BPW // CHUNK


def _emb_body(x_hbm, table_hbm, out_hbm, idx_v, rows0, rows1, sem0, sem1):
    wid = lax.axis_index("s") * NC + lax.axis_index("c")
    wbase = wid * BPW
    pltpu.sync_copy(x_hbm.at[pl.ds(wbase, BPW)], idx_v)

    def fire(g, buf, sem):
        for j in range(NGATH):
            pltpu.async_copy(
                table_hbm.at[idx_v.at[pl.ds(g * CHUNK + j * NIDX, NIDX)]],
                buf.at[pl.ds(j * NIDX, NIDX)],
                sem,
            )

    def drain_write(g, buf, sem):
        pltpu.make_async_copy(table_hbm.at[pl.ds(0, CHUNK)], buf, sem).wait()
        pltpu.sync_copy(buf, out_hbm.at[pl.ds(wbase + g * CHUNK, CHUNK)])

    fire(0, rows0, sem0)

    def body(i2, carry):
        g0 = 2 * i2
        fire(g0 + 1, rows1, sem1)
        drain_write(g0, rows0, sem0)
        fire(g0 + 2, rows0, sem0)
        drain_write(g0 + 1, rows1, sem1)
        return carry

    lax.fori_loop(0, NCHUNKS // 2 - 1, body, 0)
    g0 = NCHUNKS - 2
    fire(g0 + 1, rows1, sem1)
    drain_write(g0, rows0, sem0)
    drain_write(g0 + 1, rows1, sem1)


def kernel(x, table):
    xf = x.reshape(-1)
    table128 = jnp.pad(table, ((0, 0), (0, 128 - D)))
    mesh = plsc.VectorSubcoreMesh(core_axis_name="c", subcore_axis_name="s")
    out = pl.kernel(
        _emb_body,
        out_type=jax.ShapeDtypeStruct((B, 128), jnp.float32),
        mesh=mesh,
        scratch_types=[
            pltpu.VMEM((BPW,), jnp.int32),
            pltpu.VMEM((CHUNK, 128), jnp.float32),
            pltpu.VMEM((CHUNK, 128), jnp.float32),
            pltpu.SemaphoreType.DMA,
            pltpu.SemaphoreType.DMA,
        ],
        compiler_params=pltpu.CompilerParams(use_tc_tiling_on_sc=True),
    )(xf, table128)
    return out[:, :D].reshape(x.shape + (D,))
